# Initial kernel scaffold; baseline (speedup 1.0000x reference)
import jax
import jax.numpy as jnp
from jax.experimental import pallas as pl


def kernel(arr, mask):
    raise NotImplementedError("write your pallas kernel here")


# speedup vs baseline: 1.1972x; 1.1972x over previous
import functools

import jax
import jax.numpy as jnp
from jax import lax
from jax.experimental import pallas as pl
from jax.experimental.pallas import tpu as pltpu
from jax.experimental.pallas import tpu_sc as plsc

VOCAB = 1000
ROWS = 4096 * 20
NC, NS = 2, 16
NW = NC * NS
ROWS_PER_W = ROWS // NW
C = 32
NCHUNK = ROWS_PER_W // C
NBUF = 2
BUF_WORDS = C * VOCAB

_mesh = plsc.VectorSubcoreMesh(core_axis_name="c", subcore_axis_name="s")


@functools.partial(
    pl.kernel,
    mesh=_mesh,
    out_type=jax.ShapeDtypeStruct((ROWS * VOCAB,), jnp.float32),
    scratch_types=[
        pltpu.VMEM((ROWS_PER_W,), jnp.int32),
        pltpu.VMEM((BUF_WORDS,), jnp.float32),
        pltpu.VMEM((BUF_WORDS,), jnp.float32),
        pltpu.SemaphoreType.DMA,
        pltpu.SemaphoreType.DMA,
    ],
    compiler_params=pltpu.CompilerParams(needs_layout_passes=False),
)
def _onehot_sc(arr_hbm, out_hbm, idxs, buf0, buf1, sem0, sem1):
    wid = lax.axis_index("s") * NC + lax.axis_index("c")
    row0 = wid * ROWS_PER_W

    pltpu.sync_copy(arr_hbm.at[pl.ds(row0, ROWS_PER_W)], idxs)

    bufs = (buf0, buf1)
    sems = (sem0, sem1)

    zeros16 = jnp.zeros((16,), jnp.float32)

    def _zfill(j, carry):
        buf0[pl.ds(j * 16, 16)] = zeros16
        buf1[pl.ds(j * 16, 16)] = zeros16
        return carry

    lax.fori_loop(0, BUF_WORDS // 16, _zfill, 0)

    iota16 = lax.iota(jnp.int32, 16)
    ones16 = jnp.ones((16,), jnp.float32)

    def _scatter(buf, g, val):
        for i in range(C // 16):
            cols = idxs[pl.ds(g * C + i * 16, 16)]
            flat = (iota16 + i * 16) * VOCAB + cols
            plsc.store_scatter(buf, [flat], val)

    def _start_out(b, g):
        off = (row0 + g * C) * VOCAB
        pltpu.make_async_copy(
            bufs[b], out_hbm.at[pl.ds(off, BUF_WORDS)], sems[b]
        ).start()

    def _wait_out(b):
        pltpu.make_async_copy(
            bufs[b], out_hbm.at[pl.ds(0, BUF_WORDS)], sems[b]
        ).wait()

    for b in range(NBUF):
        _scatter(bufs[b], b, ones16)
        _start_out(b, b)

    def _chunk_body(k, carry):
        g0 = NBUF + k * NBUF
        for b in range(NBUF):
            g = g0 + b
            _wait_out(b)
            _scatter(bufs[b], g - NBUF, zeros16)
            _scatter(bufs[b], g, ones16)
            _start_out(b, g)
        return carry

    lax.fori_loop(0, (NCHUNK - NBUF) // NBUF, _chunk_body, 0)

    for b in range(NBUF):
        _wait_out(b)


def kernel(arr, mask):
    del mask
    flat = arr.reshape(-1).astype(jnp.int32)
    out = _onehot_sc(flat)
    return out.reshape(arr.shape + (VOCAB,))


# speedup vs baseline: 1.7341x; 1.4485x over previous
import functools

import jax
import jax.numpy as jnp
from jax import lax
from jax.experimental import pallas as pl
from jax.experimental.pallas import tpu as pltpu
from jax.experimental.pallas import tpu_sc as plsc

VOCAB = 1000
BATCH = 4096
HIST = 20
NC, NS = 2, 16
NW = NC * NS
B_PER_W = BATCH // NW
CB = 2
NCHUNK = B_PER_W // CB
NBUF = 2
CROWS = CB * HIST
NVEC = (CROWS + 15) // 16

_mesh = plsc.VectorSubcoreMesh(core_axis_name="c", subcore_axis_name="s")


@functools.partial(
    pl.kernel,
    mesh=_mesh,
    out_type=jax.ShapeDtypeStruct((BATCH, HIST, VOCAB), jnp.float32),
    scratch_types=[
        pltpu.VMEM((B_PER_W * HIST,), jnp.int32),
        pltpu.VMEM((CB, HIST, VOCAB), jnp.float32),
        pltpu.VMEM((CB, HIST, VOCAB), jnp.float32),
        pltpu.SemaphoreType.DMA,
        pltpu.SemaphoreType.DMA,
    ],
    compiler_params=pltpu.CompilerParams(needs_layout_passes=False),
)
def _onehot_sc(arr_hbm, out_hbm, idxs, buf0, buf1, sem0, sem1):
    wid = lax.axis_index("s") * NC + lax.axis_index("c")
    b0 = wid * B_PER_W

    pltpu.sync_copy(arr_hbm.at[pl.ds(b0 * HIST, B_PER_W * HIST)], idxs)

    bufs = (buf0, buf1)
    sems = (sem0, sem1)

    zeros16 = jnp.zeros((16,), jnp.float32)
    words = CB * HIST * VOCAB

    def _zfill(j, carry):
        r = j * 16 // VOCAB
        c = j * 16 % VOCAB
        buf0[r // HIST, r % HIST, pl.ds(c, 16)] = zeros16
        buf1[r // HIST, r % HIST, pl.ds(c, 16)] = zeros16
        return carry

    lax.fori_loop(0, words // 16, _zfill, 0)

    iota16 = lax.iota(jnp.int32, 16)
    ones16 = jnp.ones((16,), jnp.float32)

    def _scatter(buf, g, val):
        for i in range(NVEC):
            r = iota16 + i * 16
            cols = idxs[pl.ds(g * CROWS + i * 16, 16)]
            if (i + 1) * 16 <= CROWS:
                plsc.store_scatter(buf, [r // HIST, r % HIST, cols], val)
            else:
                m = r < CROWS
                plsc.store_scatter(buf, [r // HIST, r % HIST, cols], val, mask=m)

    def _start_out(b, g):
        pltpu.make_async_copy(
            bufs[b], out_hbm.at[pl.ds(b0 + g * CB, CB)], sems[b]
        ).start()

    def _wait_out(b):
        pltpu.make_async_copy(
            bufs[b], out_hbm.at[pl.ds(0, CB)], sems[b]
        ).wait()

    for b in range(NBUF):
        _scatter(bufs[b], b, ones16)
        _start_out(b, b)

    def _chunk_body(k, carry):
        g0 = NBUF + k * NBUF
        for b in range(NBUF):
            g = g0 + b
            _wait_out(b)
            _scatter(bufs[b], g - NBUF, zeros16)
            _scatter(bufs[b], g, ones16)
            _start_out(b, g)
        return carry

    lax.fori_loop(0, (NCHUNK - NBUF) // NBUF, _chunk_body, 0)

    for b in range(NBUF):
        _wait_out(b)


def kernel(arr, mask):
    del mask
    flat = arr.reshape(-1).astype(jnp.int32)
    return _onehot_sc(flat)


# speedup vs baseline: 6.5297x; 3.7656x over previous
import functools

import jax
import jax.numpy as jnp
from jax import lax
from jax.experimental import pallas as pl
from jax.experimental.pallas import tpu as pltpu
from jax.experimental.pallas import tpu_sc as plsc

VOCAB = 1000
BATCH = 4096
HIST = 20
NC, NS = 2, 16
NW = NC * NS
BW = BATCH // NW
VC = 200
VCHUNKS = VOCAB // VC
NCHUNK = HIST * VCHUNKS
NBUF = 4

_mesh = plsc.VectorSubcoreMesh(core_axis_name="c", subcore_axis_name="s")


@functools.partial(
    pl.kernel,
    mesh=_mesh,
    out_type=jax.ShapeDtypeStruct((HIST, VOCAB, BATCH), jnp.float32),
    scratch_types=[
        pltpu.VMEM((HIST, BW), jnp.int32),
        [pltpu.VMEM((VC, BW), jnp.float32)] * NBUF,
        [pltpu.SemaphoreType.DMA] * NBUF,
    ],
    compiler_params=pltpu.CompilerParams(needs_layout_passes=False),
)
def _onehot_sc(arrt_hbm, out_hbm, tcol, bufs, sems):
    wid = lax.axis_index("s") * NC + lax.axis_index("c")
    b0 = wid * BW

    pltpu.sync_copy(arrt_hbm.at[:, wid], tcol)

    zeros16 = jnp.zeros((16,), jnp.float32)

    def _zfill(j, carry):
        r = j * 16 // BW
        c = j * 16 % BW
        for b in range(NBUF):
            bufs[b][r, pl.ds(c, 16)] = zeros16
        return carry

    lax.fori_loop(0, VC * BW // 16, _zfill, 0)

    iota16 = lax.iota(jnp.int32, 16)
    ones16 = jnp.ones((16,), jnp.float32)

    def _scatter(buf, g, val):
        t = g // VCHUNKS
        v0 = (g % VCHUNKS) * VC
        for i in range(BW // 16):
            cols = tcol[t, pl.ds(i * 16, 16)]
            m = (cols >= v0) & (cols < v0 + VC)
            vloc = jnp.where(m, cols - v0, 0)
            plsc.store_scatter(buf, [vloc, iota16 + i * 16], val, mask=m)

    def _start_out(b, g):
        t = g // VCHUNKS
        v0 = (g % VCHUNKS) * VC
        pltpu.make_async_copy(
            bufs[b], out_hbm.at[t, pl.ds(v0, VC), pl.ds(b0, BW)], sems[b]
        ).start()

    def _wait_out(b):
        pltpu.make_async_copy(
            bufs[b], out_hbm.at[0, pl.ds(0, VC), pl.ds(0, BW)], sems[b]
        ).wait()

    for b in range(NBUF):
        _scatter(bufs[b], b, ones16)
        _start_out(b, b)

    def _chunk_body(k, carry):
        g0 = NBUF + k * NBUF
        for b in range(NBUF):
            g = g0 + b
            _wait_out(b)
            _scatter(bufs[b], g - NBUF, zeros16)
            _scatter(bufs[b], g, ones16)
            _start_out(b, g)
        return carry

    lax.fori_loop(0, (NCHUNK - NBUF) // NBUF, _chunk_body, 0)

    for b in range(NBUF):
        _wait_out(b)


def kernel(arr, mask):
    del mask
    arrt = jnp.transpose(arr.astype(jnp.int32), (1, 0)).reshape(HIST, NW, BW)
    out3 = _onehot_sc(arrt)
    return jnp.transpose(out3, (2, 0, 1))
